# baseline (speedup 1.0000x reference)
import functools

import jax
import jax.numpy as jnp
import numpy as np
from jax import lax
from jax.experimental import pallas as pl
from jax.experimental.pallas import tpu as pltpu
from jax.experimental.pallas import tpu_sc as plsc

_IN_DIM = 128
_BATCH = 4096
_NPAIR = _IN_DIM * (_IN_DIM + 1) // 2
_LANES = 16
_CHUNKS = _NPAIR // _LANES

_NC = 2
_NS = 16
_NW = _NC * _NS
_ROWS_PER_W = _BATCH // _NW

_i0_np, _i1_np = np.triu_indices(_IN_DIM, k=0)


def _body(x_hbm, i0_hbm, i1_hbm, out_hbm, idx0_v, idx1_v, xs_v, ob_v, sem):
    wid = lax.axis_index("s") * _NC + lax.axis_index("c")
    base = wid * _ROWS_PER_W

    pltpu.sync_copy(i0_hbm, idx0_v)
    pltpu.sync_copy(i1_hbm, idx1_v)
    pltpu.sync_copy(x_hbm.at[pl.ds(base * _IN_DIM, _ROWS_PER_W * _IN_DIM)], xs_v)

    def row_body(r, _):
        rb = jnp.full((_LANES,), r * _IN_DIM, dtype=jnp.int32)

        def chunk_body(c, _):
            k0 = idx0_v[pl.ds(c * _LANES, _LANES)]
            k1 = idx1_v[pl.ds(c * _LANES, _LANES)]
            v0 = plsc.load_gather(xs_v, [k0 + rb])
            v1 = plsc.load_gather(xs_v, [k1 + rb])
            ob_v[pl.ds(c * _LANES, _LANES)] = v0 + v1
            return ()

        lax.fori_loop(0, _CHUNKS, chunk_body, (), unroll=4)
        pltpu.sync_copy(ob_v, out_hbm.at[base + r])
        return ()

    lax.fori_loop(0, _ROWS_PER_W, row_body, ())


@jax.jit
def kernel(x):
    i0 = jnp.asarray(_i0_np, dtype=jnp.int32)
    i1 = jnp.asarray(_i1_np, dtype=jnp.int32)
    k = pl.kernel(
        _body,
        out_type=jax.ShapeDtypeStruct((_BATCH, _NPAIR), jnp.float32),
        mesh=plsc.VectorSubcoreMesh(
            core_axis_name="c", subcore_axis_name="s",
            num_cores=_NC, num_subcores=_NS,
        ),
        scratch_types=[
            pltpu.VMEM((_NPAIR,), jnp.int32),
            pltpu.VMEM((_NPAIR,), jnp.int32),
            pltpu.VMEM((_ROWS_PER_W * _IN_DIM,), jnp.float32),
            pltpu.VMEM((_NPAIR,), jnp.float32),
            pltpu.SemaphoreType.DMA,
        ],
        compiler_params=pltpu.CompilerParams(needs_layout_passes=False),
    )
    return k(x.reshape(-1), i0, i1)


# speedup vs baseline: 1.5316x; 1.5316x over previous
import jax
import jax.numpy as jnp
import numpy as np
from jax import lax
from jax.experimental import pallas as pl
from jax.experimental.pallas import tpu as pltpu
from jax.experimental.pallas import tpu_sc as plsc

_IN_DIM = 128
_BATCH = 4096
_NPAIR = _IN_DIM * (_IN_DIM + 1) // 2
_LANES = 16
_CHUNKS = _NPAIR // _LANES

_NC = 2
_NS = 16
_NW = _NC * _NS
_ROWS_PER_W = _BATCH // _NW

_RT = 4
_NT = _ROWS_PER_W // _RT

_i0_np, _i1_np = np.triu_indices(_IN_DIM, k=0)


def _body(x_hbm, i0_hbm, i1_hbm, out_hbm,
          idx0_v, idx1_v, xs_v, ob_v, sem0, sem1):
    wid = lax.axis_index("s") * _NC + lax.axis_index("c")
    base = wid * _ROWS_PER_W

    pltpu.sync_copy(i0_hbm, idx0_v)
    pltpu.sync_copy(i1_hbm, idx1_v)
    pltpu.sync_copy(x_hbm.at[pl.ds(base * _IN_DIM, _ROWS_PER_W * _IN_DIM)], xs_v)

    sems = (sem0, sem1)

    def block_body(tt, _):
        for b in range(2):
            t = tt * 2 + b
            dst = out_hbm.at[pl.ds(base + t * _RT, _RT)]

            @pl.when(tt > 0)
            def _():
                pltpu.make_async_copy(ob_v.at[b], dst, sems[b]).wait()

            def chunk_body(c, _):
                off = c * _LANES
                k0 = idx0_v[pl.ds(off, _LANES)]
                k1 = idx1_v[pl.ds(off, _LANES)]
                for rr in range(_RT):
                    xrow = xs_v.at[pl.ds((t * _RT + rr) * _IN_DIM, _IN_DIM)]
                    v0 = plsc.load_gather(xrow, [k0])
                    v1 = plsc.load_gather(xrow, [k1])
                    ob_v[b, rr, pl.ds(off, _LANES)] = v0 + v1
                return ()

            lax.fori_loop(0, _CHUNKS, chunk_body, (), unroll=2)
            pltpu.async_copy(ob_v.at[b], dst, sems[b])
        return ()

    lax.fori_loop(0, _NT // 2, block_body, ())

    for b in range(2):
        t = _NT - 2 + b
        dst = out_hbm.at[pl.ds(base + t * _RT, _RT)]
        pltpu.make_async_copy(ob_v.at[b], dst, sems[b]).wait()


@jax.jit
def kernel(x):
    i0 = jnp.asarray(_i0_np, dtype=jnp.int32)
    i1 = jnp.asarray(_i1_np, dtype=jnp.int32)
    k = pl.kernel(
        _body,
        out_type=jax.ShapeDtypeStruct((_BATCH, _NPAIR), jnp.float32),
        mesh=plsc.VectorSubcoreMesh(
            core_axis_name="c", subcore_axis_name="s",
            num_cores=_NC, num_subcores=_NS,
        ),
        scratch_types=[
            pltpu.VMEM((_NPAIR,), jnp.int32),
            pltpu.VMEM((_NPAIR,), jnp.int32),
            pltpu.VMEM((_ROWS_PER_W * _IN_DIM,), jnp.float32),
            pltpu.VMEM((2, _RT, _NPAIR), jnp.float32),
            pltpu.SemaphoreType.DMA,
            pltpu.SemaphoreType.DMA,
        ],
        compiler_params=pltpu.CompilerParams(needs_layout_passes=False),
    )
    return k(x.reshape(-1), i0, i1)


# speedup vs baseline: 2.4688x; 1.6119x over previous
import jax
import jax.numpy as jnp
import numpy as np
from jax import lax
from jax.experimental import pallas as pl
from jax.experimental.pallas import tpu as pltpu
from jax.experimental.pallas import tpu_sc as plsc

_IN_DIM = 128
_BATCH = 4096
_NPAIR = _IN_DIM * (_IN_DIM + 1) // 2
_LANES = 16
_CHUNKS = _NPAIR // _LANES

_NC = 2
_NS = 16
_NW = _NC * _NS
_ROWS_PER_W = _BATCH // _NW

_RT = 4
_NT = _ROWS_PER_W // _RT

_i0_np, _i1_np = np.triu_indices(_IN_DIM, k=0)


def _body(x_hbm, i0_hbm, i1_hbm, out_hbm,
          idx0_v, idx1_v, xs_v, ob_v, sem0, sem1):
    wid = lax.axis_index("s") * _NC + lax.axis_index("c")
    base = wid * _ROWS_PER_W

    pltpu.sync_copy(i0_hbm, idx0_v)
    pltpu.sync_copy(i1_hbm, idx1_v)
    pltpu.sync_copy(x_hbm.at[pl.ds(base * _IN_DIM, _ROWS_PER_W * _IN_DIM)], xs_v)

    sems = (sem0, sem1)

    def block_body(tt, _):
        for b in range(2):
            t = tt * 2 + b
            dst = out_hbm.at[pl.ds(base + t * _RT, _RT)]

            @pl.when(tt > 0)
            def _():
                pltpu.make_async_copy(ob_v.at[b], dst, sems[b]).wait()

            def chunk_body(c, _):
                off = c * _LANES
                k0 = idx0_v[pl.ds(off, _LANES)]
                k1 = idx1_v[pl.ds(off, _LANES)]
                g = []
                for rr in range(_RT):
                    xrow = xs_v.at[pl.ds((t * _RT + rr) * _IN_DIM, _IN_DIM)]
                    g.append((plsc.load_gather(xrow, [k0]),
                              plsc.load_gather(xrow, [k1])))
                for rr in range(_RT):
                    ob_v[b, rr, pl.ds(off, _LANES)] = g[rr][0] + g[rr][1]
                return ()

            lax.fori_loop(0, _CHUNKS, chunk_body, (), unroll=2)
            pltpu.async_copy(ob_v.at[b], dst, sems[b])
        return ()

    lax.fori_loop(0, _NT // 2, block_body, ())

    for b in range(2):
        t = _NT - 2 + b
        dst = out_hbm.at[pl.ds(base + t * _RT, _RT)]
        pltpu.make_async_copy(ob_v.at[b], dst, sems[b]).wait()


@jax.jit
def kernel(x):
    i0 = jnp.asarray(_i0_np, dtype=jnp.int32)
    i1 = jnp.asarray(_i1_np, dtype=jnp.int32)
    k = pl.kernel(
        _body,
        out_type=jax.ShapeDtypeStruct((_BATCH, _NPAIR), jnp.float32),
        mesh=plsc.VectorSubcoreMesh(
            core_axis_name="c", subcore_axis_name="s",
            num_cores=_NC, num_subcores=_NS,
        ),
        scratch_types=[
            pltpu.VMEM((_NPAIR,), jnp.int32),
            pltpu.VMEM((_NPAIR,), jnp.int32),
            pltpu.VMEM((_ROWS_PER_W * _IN_DIM,), jnp.float32),
            pltpu.VMEM((2, _RT, _NPAIR), jnp.float32),
            pltpu.SemaphoreType.DMA,
            pltpu.SemaphoreType.DMA,
        ],
        compiler_params=pltpu.CompilerParams(needs_layout_passes=False),
    )
    return k(x.reshape(-1), i0, i1)
